# baseline (speedup 1.0000x reference)
import jax
import jax.numpy as jnp
from jax import lax
from jax.experimental import pallas as pl
from jax.experimental.pallas import tpu as pltpu
from jax.experimental.pallas import tpu_sc as plsc

_B, _C, _H, _W = 16, 384, 32, 32
_E = 64
_N = _B * _H * _W
_K = 8192
_TILE = 256
_GRID = _N // _TILE


def _vq_body(z_ref, wt_ref, b_ref, emb_ref, ze_ref, idx_ref, diff_ref,
             s2_ref, cols_ref):
    i = pl.program_id(0)

    @pl.when(i == 0)
    def _():
        emb = emb_ref[...]
        s2 = jnp.sum(emb * emb, axis=1)
        s2_ref[...] = s2[None, :]
        cols_ref[...] = lax.broadcasted_iota(
            jnp.int32, (1, _K), 1).astype(jnp.float32)
        diff_ref[0, 0] = 0.0

    ze = lax.dot_general(z_ref[...], wt_ref[...], (((1,), (0,)), ((), ())),
                         preferred_element_type=jnp.float32)
    ze = ze + b_ref[...]
    ze_ref[...] = ze
    s1 = jnp.sum(ze * ze, axis=1, keepdims=True)
    m2 = lax.dot_general(ze + ze, emb_ref[...], (((1,), (1,)), ((), ())),
                         preferred_element_type=jnp.float32)
    dist = (s1 - m2) + s2_ref[...]
    mn = jnp.min(dist, axis=1, keepdims=True)
    idx_ref[...] = jnp.argmin(dist, axis=1).astype(jnp.int32)[:, None]
    diff_ref[0, 0] += jnp.sum(mn)

    @pl.when(i == _GRID - 1)
    def _():
        diff_ref[0, 0] = diff_ref[0, 0] / float(_N * _E)


def _tc_vq(z_flat, wt, b2, embed):
    return pl.pallas_call(
        _vq_body,
        grid=(_GRID,),
        in_specs=[
            pl.BlockSpec((_TILE, _C), lambda i: (i, 0)),
            pl.BlockSpec((_C, _E), lambda i: (0, 0)),
            pl.BlockSpec((1, _E), lambda i: (0, 0)),
            pl.BlockSpec((_K, _E), lambda i: (0, 0)),
        ],
        out_specs=[
            pl.BlockSpec((_TILE, _E), lambda i: (i, 0)),
            pl.BlockSpec((_TILE, 1), lambda i: (i, 0)),
            pl.BlockSpec(memory_space=pltpu.SMEM),
        ],
        out_shape=[
            jax.ShapeDtypeStruct((_N, _E), jnp.float32),
            jax.ShapeDtypeStruct((_N, 1), jnp.int32),
            jax.ShapeDtypeStruct((1, 1), jnp.float32),
        ],
        scratch_shapes=[
            pltpu.VMEM((1, _K), jnp.float32),
            pltpu.VMEM((1, _K), jnp.float32),
        ],
    )(z_flat, wt, b2, embed)


_NW = 32
_BPW = _N // _NW


def _st_body(emb_hbm, idx_hbm, ze_hbm, out_hbm, idx_v, rows_v, ze_v, sem):
    wid = lax.axis_index("s") * 2 + lax.axis_index("c")
    base = wid * _BPW
    pltpu.sync_copy(idx_hbm.at[pl.ds(base, _BPW)], idx_v)
    pltpu.async_copy(emb_hbm.at[idx_v], rows_v, sem).wait()
    pltpu.sync_copy(ze_hbm.at[pl.ds(base * _E, _BPW * _E)], ze_v)

    def row16(rb, carry):
        r0 = rb * 16
        for rr in range(16):
            for j in range(_E // 16):
                q = rows_v[r0 + rr, pl.ds(j * 16, 16)]
                v = ze_v[pl.ds((r0 + rr) * _E + j * 16, 16)]
                ze_v[pl.ds((r0 + rr) * _E + j * 16, 16)] = v + (q - v)
        return carry

    lax.fori_loop(0, _BPW // 16, row16, 0)
    pltpu.sync_copy(ze_v, out_hbm.at[pl.ds(base * _E, _BPW * _E)])


def _st_gather(embed, idx, ze_flat):
    mesh = plsc.VectorSubcoreMesh(core_axis_name="c", subcore_axis_name="s")
    fn = pl.kernel(
        _st_body,
        mesh=mesh,
        out_type=jax.ShapeDtypeStruct((_N * _E,), jnp.float32),
        scratch_types=[
            pltpu.VMEM((_BPW,), jnp.int32),
            pltpu.VMEM((_BPW, 128), jnp.float32),
            pltpu.VMEM((_BPW * _E,), jnp.float32),
            pltpu.SemaphoreType.DMA,
        ],
    )
    return fn(embed, idx, ze_flat)


def kernel(z, W_proj, b_proj, embed):
    z_flat = z.transpose(0, 2, 3, 1).reshape(_N, _C)
    wt = W_proj.T
    b2 = b_proj.reshape(1, _E)
    ze, idx2, diff = _tc_vq(z_flat, wt, b2, embed)
    idx = idx2.reshape(_N)
    emb_pad = jnp.pad(embed, ((0, 0), (0, 128 - _E)))
    zq_st = _st_gather(emb_pad, idx, ze.reshape(_N * _E))
    z_q_st = zq_st.reshape(_B, _H, _W, _E)
    embed_ind = idx.reshape(_B, _H, _W)
    return z_q_st, diff.reshape(()), embed_ind


# speedup vs baseline: 1.0392x; 1.0392x over previous
import jax
import jax.numpy as jnp
from jax import lax
from jax.experimental import pallas as pl
from jax.experimental.pallas import tpu as pltpu
from jax.experimental.pallas import tpu_sc as plsc

_B, _C, _H, _W = 16, 384, 32, 32
_E = 64
_N = _B * _H * _W
_K = 8192
_TILE = 512
_GRID = _N // _TILE


def _vq_body(z_ref, wt_ref, b_ref, emb_ref, ze_ref, idx_ref, diff_ref,
             s2_ref, cols_ref):
    i = pl.program_id(0)

    @pl.when(i == 0)
    def _():
        emb = emb_ref[...]
        s2 = jnp.sum(emb * emb, axis=1)
        s2_ref[...] = s2[None, :]
        cols_ref[...] = lax.broadcasted_iota(
            jnp.int32, (1, _K), 1).astype(jnp.float32)
        diff_ref[0, 0] = 0.0

    ze = lax.dot_general(z_ref[...], wt_ref[...], (((1,), (0,)), ((), ())),
                         preferred_element_type=jnp.float32)
    ze = ze + b_ref[...]
    ze_ref[...] = ze
    s1 = jnp.sum(ze * ze, axis=1, keepdims=True)
    m2 = lax.dot_general(ze + ze, emb_ref[...], (((1,), (1,)), ((), ())),
                         preferred_element_type=jnp.float32)
    dist = (s1 - m2) + s2_ref[...]
    mn = jnp.min(dist, axis=1, keepdims=True)
    idxf = jnp.min(jnp.where(dist == mn, cols_ref[...], jnp.float32(_K)),
                   axis=1, keepdims=True)
    idx_ref[...] = idxf.astype(jnp.int32)
    diff_ref[0, 0] += jnp.sum(mn)

    @pl.when(i == _GRID - 1)
    def _():
        diff_ref[0, 0] = diff_ref[0, 0] / float(_N * _E)


def _tc_vq(z_flat, wt, b2, embed):
    return pl.pallas_call(
        _vq_body,
        grid=(_GRID,),
        in_specs=[
            pl.BlockSpec((_TILE, _C), lambda i: (i, 0)),
            pl.BlockSpec((_C, _E), lambda i: (0, 0)),
            pl.BlockSpec((1, _E), lambda i: (0, 0)),
            pl.BlockSpec((_K, _E), lambda i: (0, 0)),
        ],
        out_specs=[
            pl.BlockSpec((_TILE, _E), lambda i: (i, 0)),
            pl.BlockSpec((_TILE, 1), lambda i: (i, 0)),
            pl.BlockSpec(memory_space=pltpu.SMEM),
        ],
        out_shape=[
            jax.ShapeDtypeStruct((_N, _E), jnp.float32),
            jax.ShapeDtypeStruct((_N, 1), jnp.int32),
            jax.ShapeDtypeStruct((1, 1), jnp.float32),
        ],
        scratch_shapes=[
            pltpu.VMEM((1, _K), jnp.float32),
            pltpu.VMEM((1, _K), jnp.float32),
        ],
    )(z_flat, wt, b2, embed)


_NW = 32
_BPW = _N // _NW


def _st_body(emb_hbm, idx_hbm, ze_hbm, out_hbm, idx_v, rows_v, ze_v, sem):
    wid = lax.axis_index("s") * 2 + lax.axis_index("c")
    base = wid * _BPW
    pltpu.sync_copy(idx_hbm.at[pl.ds(base, _BPW)], idx_v)
    pltpu.async_copy(emb_hbm.at[idx_v], rows_v, sem).wait()
    pltpu.sync_copy(ze_hbm.at[pl.ds(base * _E, _BPW * _E)], ze_v)

    def row16(rb, carry):
        r0 = rb * 16
        for rr in range(16):
            for j in range(_E // 16):
                q = rows_v[r0 + rr, pl.ds(j * 16, 16)]
                v = ze_v[pl.ds((r0 + rr) * _E + j * 16, 16)]
                ze_v[pl.ds((r0 + rr) * _E + j * 16, 16)] = v + (q - v)
        return carry

    lax.fori_loop(0, _BPW // 16, row16, 0)
    pltpu.sync_copy(ze_v, out_hbm.at[pl.ds(base * _E, _BPW * _E)])


def _st_gather(embed, idx, ze_flat):
    mesh = plsc.VectorSubcoreMesh(core_axis_name="c", subcore_axis_name="s")
    fn = pl.kernel(
        _st_body,
        mesh=mesh,
        out_type=jax.ShapeDtypeStruct((_N * _E,), jnp.float32),
        scratch_types=[
            pltpu.VMEM((_BPW,), jnp.int32),
            pltpu.VMEM((_BPW, 128), jnp.float32),
            pltpu.VMEM((_BPW * _E,), jnp.float32),
            pltpu.SemaphoreType.DMA,
        ],
    )
    return fn(embed, idx, ze_flat)


def kernel(z, W_proj, b_proj, embed):
    z_flat = z.transpose(0, 2, 3, 1).reshape(_N, _C)
    wt = W_proj.T
    b2 = b_proj.reshape(1, _E)
    ze, idx2, diff = _tc_vq(z_flat, wt, b2, embed)
    idx = idx2.reshape(_N)
    emb_pad = jnp.pad(embed, ((0, 0), (0, 128 - _E)))
    zq_st = _st_gather(emb_pad, idx, ze.reshape(_N * _E))
    z_q_st = zq_st.reshape(_B, _H, _W, _E)
    embed_ind = idx.reshape(_B, _H, _W)
    return z_q_st, diff.reshape(()), embed_ind


# speedup vs baseline: 1.1286x; 1.0861x over previous
import jax
import jax.numpy as jnp
from jax import lax
from jax.experimental import pallas as pl
from jax.experimental.pallas import tpu as pltpu
from jax.experimental.pallas import tpu_sc as plsc

_B, _C, _H, _W = 16, 384, 32, 32
_E = 64
_N = _B * _H * _W
_K = 8192
_TILE = 1024
_GRID = _N // _TILE


def _vq_body(z_ref, wt_ref, b_ref, emb_ref, ze_ref, idx_ref, diff_ref,
             s2_ref, cols_ref):
    i = pl.program_id(0)

    @pl.when(i == 0)
    def _():
        emb = emb_ref[...]
        s2 = jnp.sum(emb * emb, axis=1)
        s2_ref[...] = s2[None, :]
        cols_ref[...] = lax.broadcasted_iota(
            jnp.int32, (1, _K), 1).astype(jnp.float32)
        diff_ref[0, 0] = 0.0

    ze = lax.dot_general(z_ref[...], wt_ref[...], (((1,), (0,)), ((), ())),
                         preferred_element_type=jnp.float32)
    ze = ze + b_ref[...]
    ze_ref[...] = ze
    s1 = jnp.sum(ze * ze, axis=1, keepdims=True)
    m2 = lax.dot_general(ze + ze, emb_ref[...], (((1,), (1,)), ((), ())),
                         preferred_element_type=jnp.float32)
    dist = (s1 - m2) + s2_ref[...]
    mn = jnp.min(dist, axis=1, keepdims=True)
    idxf = jnp.min(jnp.where(dist == mn, cols_ref[...], jnp.float32(_K)),
                   axis=1, keepdims=True)
    idx_ref[...] = idxf.astype(jnp.int32)
    diff_ref[0, 0] += jnp.sum(mn)

    @pl.when(i == _GRID - 1)
    def _():
        diff_ref[0, 0] = diff_ref[0, 0] / float(_N * _E)


def _tc_vq(z_flat, wt, b2, embed):
    return pl.pallas_call(
        _vq_body,
        grid=(_GRID,),
        in_specs=[
            pl.BlockSpec((_TILE, _C), lambda i: (i, 0)),
            pl.BlockSpec((_C, _E), lambda i: (0, 0)),
            pl.BlockSpec((1, _E), lambda i: (0, 0)),
            pl.BlockSpec((_K, _E), lambda i: (0, 0)),
        ],
        out_specs=[
            pl.BlockSpec((_TILE, _E), lambda i: (i, 0)),
            pl.BlockSpec((_TILE, 1), lambda i: (i, 0)),
            pl.BlockSpec(memory_space=pltpu.SMEM),
        ],
        out_shape=[
            jax.ShapeDtypeStruct((_N, _E), jnp.float32),
            jax.ShapeDtypeStruct((_N, 1), jnp.int32),
            jax.ShapeDtypeStruct((1, 1), jnp.float32),
        ],
        scratch_shapes=[
            pltpu.VMEM((1, _K), jnp.float32),
            pltpu.VMEM((1, _K), jnp.float32),
        ],
    )(z_flat, wt, b2, embed)


_NW = 32
_BPW = _N // _NW


def _st_body(emb_hbm, idx_hbm, ze_hbm, out_hbm, idx_v, rows_v, ze_v, sem):
    wid = lax.axis_index("s") * 2 + lax.axis_index("c")
    base = wid * _BPW
    pltpu.sync_copy(idx_hbm.at[pl.ds(base, _BPW)], idx_v)
    pltpu.async_copy(emb_hbm.at[idx_v], rows_v, sem).wait()
    pltpu.sync_copy(ze_hbm.at[pl.ds(base * _E, _BPW * _E)], ze_v)

    def row16(rb, carry):
        r0 = rb * 16
        for rr in range(16):
            for j in range(_E // 16):
                q = rows_v[r0 + rr, pl.ds(j * 16, 16)]
                v = ze_v[pl.ds((r0 + rr) * _E + j * 16, 16)]
                ze_v[pl.ds((r0 + rr) * _E + j * 16, 16)] = v + (q - v)
        return carry

    lax.fori_loop(0, _BPW // 16, row16, 0)
    pltpu.sync_copy(ze_v, out_hbm.at[pl.ds(base * _E, _BPW * _E)])


def _st_gather(embed, idx, ze_flat):
    mesh = plsc.VectorSubcoreMesh(core_axis_name="c", subcore_axis_name="s")
    fn = pl.kernel(
        _st_body,
        mesh=mesh,
        out_type=jax.ShapeDtypeStruct((_N * _E,), jnp.float32),
        scratch_types=[
            pltpu.VMEM((_BPW,), jnp.int32),
            pltpu.VMEM((_BPW, 128), jnp.float32),
            pltpu.VMEM((_BPW * _E,), jnp.float32),
            pltpu.SemaphoreType.DMA,
        ],
    )
    return fn(embed, idx, ze_flat)


def kernel(z, W_proj, b_proj, embed):
    z_flat = z.transpose(0, 2, 3, 1).reshape(_N, _C)
    wt = W_proj.T
    b2 = b_proj.reshape(1, _E)
    ze, idx2, diff = _tc_vq(z_flat, wt, b2, embed)
    idx = idx2.reshape(_N)
    emb_pad = jnp.pad(embed, ((0, 0), (0, 128 - _E)))
    zq_st = _st_gather(emb_pad, idx, ze.reshape(_N * _E))
    z_q_st = zq_st.reshape(_B, _H, _W, _E)
    embed_ind = idx.reshape(_B, _H, _W)
    return z_q_st, diff.reshape(()), embed_ind
